# Initial kernel scaffold; baseline (speedup 1.0000x reference)
import functools

import jax
import jax.numpy as jnp
from jax import lax
from jax.experimental import pallas as pl
from jax.experimental.pallas import tpu as pltpu
from jax.experimental.pallas import tpu_sc as plsc

N_NODES = 10000
D = 128
KV = 16
ROT = 4
N_PAD = 10240
NW = 32
V_TOT = N_PAD * KV
CV = 64
NODES_PER_CHUNK = CV // KV
N_CHUNKS = V_TOT // CV
CPW = N_CHUNKS // NW
BN = 512


def _sc_interpolate(sig, ipk, wpk):
    mesh = plsc.VectorSubcoreMesh(
        core_axis_name="c", subcore_axis_name="s", num_cores=2, num_subcores=16)

    @functools.partial(
        pl.kernel,
        mesh=mesh,
        out_type=jax.ShapeDtypeStruct((N_PAD, KV, D), jnp.float32),
        scratch_types=[
            pltpu.VMEM((3, CV), jnp.int32),
            pltpu.VMEM((3, CV), jnp.float32),
            pltpu.VMEM((3, CV, D), jnp.float32),
            pltpu.VMEM((NODES_PER_CHUNK, KV, D), jnp.float32),
            pltpu.SemaphoreType.DMA,
        ],
    )
    def sc_kernel(sig_hbm, ipk_hbm, wpk_hbm, x_hbm, idx_v, w_v, g_v, x_v, sem):
        wid = lax.axis_index("s") * 2 + lax.axis_index("c")

        def chunk_body(ci, carry):
            g = wid * CPW + ci
            pltpu.sync_copy(ipk_hbm.at[g], idx_v)
            pltpu.sync_copy(wpk_hbm.at[g], w_v)
            cps = [pltpu.async_copy(sig_hbm.at[idx_v.at[k]], g_v.at[k], sem)
                   for k in range(3)]
            for cp in cps:
                cp.wait()

            def vert_body(t, c2):
                node = t // KV
                v = t % KV
                tvec = jnp.full((16,), t, dtype=jnp.int32)
                zvec = jnp.zeros((16,), dtype=jnp.int32)
                wa = plsc.load_gather(w_v, [zvec, tvec])
                wb = plsc.load_gather(w_v, [zvec + 1, tvec])
                wc = plsc.load_gather(w_v, [zvec + 2, tvec])
                for l in range(D // 16):
                    s = 16 * l
                    xa = (wa * g_v[0, t, pl.ds(s, 16)]
                          + wb * g_v[1, t, pl.ds(s, 16)]
                          + wc * g_v[2, t, pl.ds(s, 16)])
                    x_v[node, v, pl.ds(s, 16)] = xa
                return c2

            lax.fori_loop(0, CV, vert_body, 0, unroll=False)
            nbase = g * NODES_PER_CHUNK
            pltpu.sync_copy(x_v, x_hbm.at[pl.ds(nbase, NODES_PER_CHUNK)])
            return carry

        lax.fori_loop(0, CPW, chunk_body, 0, unroll=False)

    return sc_kernel(sig, ipk, wpk)


def _tc_body(x_ref, w_ref, o_ref):
    acc = jnp.zeros((BN, ROT * D), dtype=jnp.float32)
    for v in range(KV):
        acc = acc + jnp.dot(x_ref[:, v, :], w_ref[v],
                            preferred_element_type=jnp.float32)
    y = jnp.maximum(acc, 0.0)
    out = y[:, 0:D]
    best = jnp.sum(out * out, axis=1, keepdims=True)
    for r in range(1, ROT):
        yr = y[:, r * D:(r + 1) * D]
        nr = jnp.sum(yr * yr, axis=1, keepdims=True)
        gt = nr > best
        out = jnp.where(gt, yr, out)
        best = jnp.where(gt, nr, best)
    o_ref[...] = out


def _tc_conv(x, w3):
    return pl.pallas_call(
        _tc_body,
        grid=(N_PAD // BN,),
        in_specs=[
            pl.BlockSpec((BN, KV, D), lambda i: (i, 0, 0)),
            pl.BlockSpec((KV, D, ROT * D), lambda i: (0, 0, 0)),
        ],
        out_specs=pl.BlockSpec((BN, D), lambda i: (i, 0)),
        out_shape=jax.ShapeDtypeStruct((N_PAD, D), jnp.float32),
    )(x, w3)


def _prep_indices(bc):
    idx = bc[..., 3::2].astype(jnp.int32)
    w = bc[..., 2::2]
    idx = idx.reshape(N_NODES * KV, 3)
    w = w.reshape(N_NODES * KV, 3)
    pad = V_TOT - N_NODES * KV
    idx = jnp.pad(idx, ((0, pad), (0, 0)))
    w = jnp.pad(w, ((0, pad), (0, 0)))
    ipk = idx.reshape(N_CHUNKS, CV, 3).transpose(0, 2, 1)
    wpk = w.reshape(N_CHUNKS, CV, 3).transpose(0, 2, 1)
    return ipk, wpk


def _prep_weights(kern):
    k2 = kern[:, :, 0]
    krot = jnp.stack([jnp.roll(k2, -r, axis=1) for r in range(ROT)], axis=0)
    w3 = krot.transpose(1, 2, 4, 0, 3)
    return w3.reshape(KV, D, ROT * D)


def kernel(signal, b_coordinates, kernel):
    sig = signal[0]
    bc = b_coordinates[0]
    ipk, wpk = _prep_indices(bc)
    w3 = _prep_weights(kernel)
    x = _sc_interpolate(sig, ipk, wpk)
    out = _tc_conv(x, w3)
    return out[:N_NODES][None]


# speedup vs baseline: 3.0785x; 3.0785x over previous
import functools

import jax
import jax.numpy as jnp
from jax import lax
from jax.experimental import pallas as pl
from jax.experimental.pallas import tpu as pltpu
from jax.experimental.pallas import tpu_sc as plsc

N_NODES = 10000
D = 128
KV = 16
ROT = 4
N_PAD = 10240
NW = 32
V_TOT = N_PAD * KV
CV = 64
NODES_PER_CHUNK = CV // KV
N_CHUNKS = V_TOT // CV
CPW = N_CHUNKS // NW
BN = 512


def _sc_interpolate(sig, ipk, wpk):
    mesh = plsc.VectorSubcoreMesh(
        core_axis_name="c", subcore_axis_name="s", num_cores=2, num_subcores=16)

    @functools.partial(
        pl.kernel,
        mesh=mesh,
        out_type=jax.ShapeDtypeStruct((N_PAD, KV, D), jnp.float32),
        scratch_types=[
            pltpu.VMEM((3 * CV,), jnp.int32),
            pltpu.VMEM((3 * CV + 16,), jnp.float32),
            pltpu.VMEM((3, CV, D), jnp.float32),
            pltpu.VMEM((NODES_PER_CHUNK, KV, D), jnp.float32),
            pltpu.SemaphoreType.DMA,
        ],
    )
    def sc_kernel(sig_hbm, ipk_hbm, wpk_hbm, x_hbm, idx_v, w_v, g_v, x_v, sem):
        wid = lax.axis_index("s") * 2 + lax.axis_index("c")

        def chunk_body(ci, carry):
            g = wid * CPW + ci
            pltpu.sync_copy(ipk_hbm.at[pl.ds(g * 3 * CV, 3 * CV)], idx_v)
            pltpu.sync_copy(wpk_hbm.at[pl.ds(g * 3 * CV, 3 * CV)],
                            w_v.at[pl.ds(0, 3 * CV)])
            cps = [pltpu.async_copy(
                       sig_hbm.at[idx_v.at[pl.ds(k * CV, CV)]],
                       g_v.at[k], sem)
                   for k in range(3)]
            for cp in cps:
                cp.wait()

            def vert_body(t, c2):
                node = t // KV
                v = t % KV
                wa = jnp.full((16,), w_v[pl.ds(t, 16)][0], dtype=jnp.float32)
                wb = jnp.full((16,), w_v[pl.ds(CV + t, 16)][0],
                              dtype=jnp.float32)
                wc = jnp.full((16,), w_v[pl.ds(2 * CV + t, 16)][0],
                              dtype=jnp.float32)
                for l in range(D // 16):
                    s = 16 * l
                    xa = (wa * g_v[0, t, pl.ds(s, 16)]
                          + wb * g_v[1, t, pl.ds(s, 16)]
                          + wc * g_v[2, t, pl.ds(s, 16)])
                    x_v[node, v, pl.ds(s, 16)] = xa
                return c2

            lax.fori_loop(0, CV, vert_body, 0, unroll=False)
            nbase = g * NODES_PER_CHUNK
            pltpu.sync_copy(x_v, x_hbm.at[pl.ds(nbase, NODES_PER_CHUNK)])
            return carry

        lax.fori_loop(0, CPW, chunk_body, 0, unroll=False)

    return sc_kernel(sig, ipk, wpk)


def _tc_body(x_ref, w_ref, o_ref):
    acc = jnp.zeros((BN, ROT * D), dtype=jnp.float32)
    for v in range(KV):
        acc = acc + jnp.dot(x_ref[:, v, :], w_ref[v],
                            preferred_element_type=jnp.float32)
    y = jnp.maximum(acc, 0.0)
    out = y[:, 0:D]
    best = jnp.sum(out * out, axis=1, keepdims=True)
    for r in range(1, ROT):
        yr = y[:, r * D:(r + 1) * D]
        nr = jnp.sum(yr * yr, axis=1, keepdims=True)
        gt = nr > best
        out = jnp.where(gt, yr, out)
        best = jnp.where(gt, nr, best)
    o_ref[...] = out


def _tc_conv(x, w3):
    return pl.pallas_call(
        _tc_body,
        grid=(N_PAD // BN,),
        in_specs=[
            pl.BlockSpec((BN, KV, D), lambda i: (i, 0, 0)),
            pl.BlockSpec((KV, D, ROT * D), lambda i: (0, 0, 0)),
        ],
        out_specs=pl.BlockSpec((BN, D), lambda i: (i, 0)),
        out_shape=jax.ShapeDtypeStruct((N_PAD, D), jnp.float32),
    )(x, w3)


def _prep_indices(bc):
    idx = bc[..., 3::2].astype(jnp.int32)
    w = bc[..., 2::2]
    idx = idx.reshape(N_NODES * KV, 3)
    w = w.reshape(N_NODES * KV, 3)
    pad = V_TOT - N_NODES * KV
    idx = jnp.pad(idx, ((0, pad), (0, 0)))
    w = jnp.pad(w, ((0, pad), (0, 0)))
    ipk = idx.reshape(N_CHUNKS, CV, 3).transpose(0, 2, 1).reshape(-1)
    wpk = w.reshape(N_CHUNKS, CV, 3).transpose(0, 2, 1).reshape(-1)
    return ipk, wpk


def _prep_weights(kern):
    k2 = kern[:, :, 0]
    krot = jnp.stack([jnp.roll(k2, -r, axis=1) for r in range(ROT)], axis=0)
    w3 = krot.transpose(1, 2, 4, 0, 3)
    return w3.reshape(KV, D, ROT * D)


def kernel(signal, b_coordinates, kernel):
    sig = signal[0]
    bc = b_coordinates[0]
    ipk, wpk = _prep_indices(bc)
    w3 = _prep_weights(kernel)
    x = _sc_interpolate(sig, ipk, wpk)
    out = _tc_conv(x, w3)
    return out[:N_NODES][None]
